# baseline (speedup 1.0000x reference)
import functools

import jax
import jax.numpy as jnp
from jax import lax
from jax.experimental import pallas as pl
from jax.experimental.pallas import tpu as pltpu
from jax.experimental.pallas import tpu_sc as plsc

B = 2048
IN = 256
OUT = 128
LANE = 128
NCHUNK = B // LANE
OGRP = 8

SC_NC = 2
SC_NS = 16
SCL = 16
NW = SC_NC * SC_NS
OUT_H = OUT // 2
CPT = OUT_H // NW
NV = IN // SCL


def _score_body(xT_ref, tT_ref, out_ref):
    f32 = jnp.float32
    ones_col = jnp.ones((LANE, 1), f32)

    dacc = xT_ref[:, 0:LANE]
    for c in range(1, NCHUNK):
        dacc = dacc + xT_ref[:, c * LANE:(c + 1) * LANE]
    denomT = lax.dot_general(ones_col, dacc, (((0,), (1,)), ((), ())),
                             preferred_element_type=f32)

    def gbody(g, carry):
        o0 = g * OGRP
        for ih in range(2):
            rs = slice(ih * (IN // 2), (ih + 1) * (IN // 2))
            accs = [None] * OGRP
            for c in range(NCHUNK):
                cs = slice(c * LANE, (c + 1) * LANE)
                xc = xT_ref[rs, cs]
                t8 = tT_ref[pl.ds(o0, OGRP), cs]
                for r in range(OGRP):
                    trow = lax.slice(t8, (r, 0), (r + 1, LANE))
                    m = jnp.minimum(xc, trow)
                    accs[r] = m if c == 0 else accs[r] + m
            srows = [lax.dot_general(ones_col, accs[r], (((0,), (1,)), ((), ())),
                                     preferred_element_type=f32)
                     for r in range(OGRP)]
            sblkT = jnp.concatenate(srows, axis=0)
            out_ref[pl.ds(o0, OGRP), rs] = sblkT
        return carry

    lax.fori_loop(0, OUT_H // OGRP, gbody, 0)

    sT = out_ref[...]
    out_ref[...] = jnp.where(denomT == 0.0, jnp.float32(1.0), sT / denomT)


def _sc_clean_k(ks):
    r = len(ks)
    if r == 1:
        s, _ = plsc.sort_key_val(ks[0], ks[0], descending=True)
        return [s]
    h = r // 2
    lo = [jnp.maximum(ks[j], ks[j + h]) for j in range(h)]
    hi = [jnp.minimum(ks[j], ks[j + h]) for j in range(h)]
    return _sc_clean_k(lo) + _sc_clean_k(hi)


def _sc_merge_k(a, b):
    r = len(a)
    b = [lax.rev(x, (0,)) for x in reversed(b)]
    lo = [jnp.maximum(a[j], b[j]) for j in range(r)]
    hi = [jnp.minimum(a[j], b[j]) for j in range(r)]
    return _sc_clean_k(lo) + _sc_clean_k(hi)


def _sc_sort_col_k(ks):
    runs = [[plsc.sort_key_val(k, k, descending=True)[0]] for k in ks]
    while len(runs) > 1:
        runs = [_sc_merge_k(runs[i], runs[i + 1])
                for i in range(0, len(runs), 2)]
    return runs[0]


def _sc_clean_kv(ks, vs):
    r = len(ks)
    if r == 1:
        k, v = plsc.sort_key_val(ks[0], vs[0], descending=True)
        return [k], [v]
    h = r // 2
    lo_k, lo_v, hi_k, hi_v = [], [], [], []
    for j in range(h):
        sel = ks[j] >= ks[j + h]
        lo_k.append(jnp.where(sel, ks[j], ks[j + h]))
        lo_v.append(jnp.where(sel, vs[j], vs[j + h]))
        hi_k.append(jnp.where(sel, ks[j + h], ks[j]))
        hi_v.append(jnp.where(sel, vs[j + h], vs[j]))
    ak, av = _sc_clean_kv(lo_k, lo_v)
    bk, bv = _sc_clean_kv(hi_k, hi_v)
    return ak + bk, av + bv


def _sc_merge_kv(ak, av, bk, bv):
    r = len(ak)
    bk = [lax.rev(x, (0,)) for x in reversed(bk)]
    bv = [lax.rev(x, (0,)) for x in reversed(bv)]
    lo_k, lo_v, hi_k, hi_v = [], [], [], []
    for j in range(r):
        sel = ak[j] >= bk[j]
        lo_k.append(jnp.where(sel, ak[j], bk[j]))
        lo_v.append(jnp.where(sel, av[j], bv[j]))
        hi_k.append(jnp.where(sel, bk[j], ak[j]))
        hi_v.append(jnp.where(sel, bv[j], av[j]))
    ck, cv = _sc_clean_kv(lo_k, lo_v)
    dk, dv = _sc_clean_kv(hi_k, hi_v)
    return ck + dk, cv + dv


def _sc_sort_col_kv(ks, vs):
    runs = [tuple([x] for x in plsc.sort_key_val(k, v, descending=True))
            for k, v in zip(ks, vs)]
    while len(runs) > 1:
        runs = [_sc_merge_kv(*runs[i], *runs[i + 1])
                for i in range(0, len(runs), 2)]
    return runs[0]


def _sc_sort_body(scoreT_hbm, wT_hbm, out_hbm, sv, wv, pv):
    cid = lax.axis_index("c")
    sid = lax.axis_index("s")
    wid = sid * SC_NC + cid
    row0 = wid * CPT
    pltpu.sync_copy(scoreT_hbm.at[pl.ds(row0, CPT)], sv)
    pltpu.sync_copy(wT_hbm.at[pl.ds(row0, CPT)], wv)

    acc = jnp.zeros((SCL,), jnp.float32)
    for c in range(CPT):
        ks = [sv[c, pl.ds(v * SCL, SCL)] for v in range(NV)]
        vs = [wv[c, pl.ds(v * SCL, SCL)] for v in range(NV)]
        _, tw = _sc_sort_col_kv(ks, vs)
        sw = _sc_sort_col_k(vs)
        for v in range(NV):
            d = sw[v] - tw[v]
            acc = acc + d * d
    pv[...] = acc
    pltpu.sync_copy(pv, out_hbm.at[wid])


_sc_sort = functools.partial(
    pl.kernel,
    out_type=jax.ShapeDtypeStruct((NW, SCL), jnp.float32),
    mesh=plsc.VectorSubcoreMesh(core_axis_name="c", subcore_axis_name="s"),
    compiler_params=pltpu.CompilerParams(needs_layout_passes=False),
    scratch_types=[
        pltpu.VMEM((CPT, IN), jnp.float32),
        pltpu.VMEM((CPT, IN), jnp.float32),
        pltpu.VMEM((SCL,), jnp.float32),
    ],
)(_sc_sort_body)


@jax.jit
def _run(x, t, w):
    xT = x.T
    tT = t.T
    wT = w.T
    score0 = pl.pallas_call(
        _score_body,
        out_shape=jax.ShapeDtypeStruct((OUT_H, IN), jnp.float32),
    )(xT, tT[:OUT_H])
    p0 = _sc_sort(score0, wT[:OUT_H])
    score1 = pl.pallas_call(
        _score_body,
        out_shape=jax.ShapeDtypeStruct((OUT_H, IN), jnp.float32),
    )(xT, tT[OUT_H:])
    p1 = _sc_sort(score1, wT[OUT_H:])
    return (jnp.sum(p0) + jnp.sum(p1)) / jnp.float32(IN * OUT)


def kernel(x, y, t, w):
    del y
    return _run(x, t, w)


# speedup vs baseline: 1.0490x; 1.0490x over previous
import functools

import jax
import jax.numpy as jnp
from jax import lax
from jax.experimental import pallas as pl
from jax.experimental.pallas import tpu as pltpu
from jax.experimental.pallas import tpu_sc as plsc

B = 2048
IN = 256
OUT = 128
LANE = 128
NCHUNK = B // LANE
OGRP = 8

SC_NC = 2
SC_NS = 16
SCL = 16
NW = SC_NC * SC_NS
CPT = OUT // NW
NV = IN // SCL


def _score_body(xT_ref, tT_ref, out_ref):
    f32 = jnp.float32
    ones_col = jnp.ones((LANE, 1), f32)

    dacc = xT_ref[:, 0:LANE]
    for c in range(1, NCHUNK):
        dacc = dacc + xT_ref[:, c * LANE:(c + 1) * LANE]
    denomT = lax.dot_general(ones_col, dacc, (((0,), (1,)), ((), ())),
                             preferred_element_type=f32)

    def gbody(g, carry):
        o0 = g * OGRP
        for ih in range(2):
            rs = slice(ih * (IN // 2), (ih + 1) * (IN // 2))
            accs = [None] * OGRP
            for c in range(NCHUNK):
                cs = slice(c * LANE, (c + 1) * LANE)
                xc = xT_ref[rs, cs]
                t8 = tT_ref[pl.ds(o0, OGRP), cs]
                for r in range(OGRP):
                    trow = lax.slice(t8, (r, 0), (r + 1, LANE))
                    m = jnp.minimum(xc, trow)
                    accs[r] = m if c == 0 else accs[r] + m
            srows = [lax.dot_general(ones_col, accs[r], (((0,), (1,)), ((), ())),
                                     preferred_element_type=f32)
                     for r in range(OGRP)]
            sblkT = jnp.concatenate(srows, axis=0)
            out_ref[pl.ds(o0, OGRP), rs] = sblkT
        return carry

    lax.fori_loop(0, OUT // OGRP, gbody, 0)

    sT = out_ref[...]
    out_ref[...] = jnp.where(denomT == 0.0, jnp.float32(1.0), sT / denomT)


def _sc_clean_k(ks):
    r = len(ks)
    if r == 1:
        s, _ = plsc.sort_key_val(ks[0], ks[0], descending=True)
        return [s]
    h = r // 2
    lo = [jnp.maximum(ks[j], ks[j + h]) for j in range(h)]
    hi = [jnp.minimum(ks[j], ks[j + h]) for j in range(h)]
    return _sc_clean_k(lo) + _sc_clean_k(hi)


def _sc_merge_k(a, b):
    r = len(a)
    b = [lax.rev(x, (0,)) for x in reversed(b)]
    lo = [jnp.maximum(a[j], b[j]) for j in range(r)]
    hi = [jnp.minimum(a[j], b[j]) for j in range(r)]
    return _sc_clean_k(lo) + _sc_clean_k(hi)


def _sc_sort_col_k(ks):
    runs = [[plsc.sort_key_val(k, k, descending=True)[0]] for k in ks]
    while len(runs) > 1:
        runs = [_sc_merge_k(runs[i], runs[i + 1])
                for i in range(0, len(runs), 2)]
    return runs[0]


def _sc_clean_kv(ks, vs):
    r = len(ks)
    if r == 1:
        k, v = plsc.sort_key_val(ks[0], vs[0], descending=True)
        return [k], [v]
    h = r // 2
    lo_k, lo_v, hi_k, hi_v = [], [], [], []
    for j in range(h):
        sel = ks[j] >= ks[j + h]
        lo_k.append(jnp.where(sel, ks[j], ks[j + h]))
        lo_v.append(jnp.where(sel, vs[j], vs[j + h]))
        hi_k.append(jnp.where(sel, ks[j + h], ks[j]))
        hi_v.append(jnp.where(sel, vs[j + h], vs[j]))
    ak, av = _sc_clean_kv(lo_k, lo_v)
    bk, bv = _sc_clean_kv(hi_k, hi_v)
    return ak + bk, av + bv


def _sc_merge_kv(ak, av, bk, bv):
    r = len(ak)
    bk = [lax.rev(x, (0,)) for x in reversed(bk)]
    bv = [lax.rev(x, (0,)) for x in reversed(bv)]
    lo_k, lo_v, hi_k, hi_v = [], [], [], []
    for j in range(r):
        sel = ak[j] >= bk[j]
        lo_k.append(jnp.where(sel, ak[j], bk[j]))
        lo_v.append(jnp.where(sel, av[j], bv[j]))
        hi_k.append(jnp.where(sel, bk[j], ak[j]))
        hi_v.append(jnp.where(sel, bv[j], av[j]))
    ck, cv = _sc_clean_kv(lo_k, lo_v)
    dk, dv = _sc_clean_kv(hi_k, hi_v)
    return ck + dk, cv + dv


def _sc_sort_col_kv(ks, vs):
    runs = [tuple([x] for x in plsc.sort_key_val(k, v, descending=True))
            for k, v in zip(ks, vs)]
    while len(runs) > 1:
        runs = [_sc_merge_kv(*runs[i], *runs[i + 1])
                for i in range(0, len(runs), 2)]
    return runs[0]


def _sc_sort_body(scoreT_hbm, wT_hbm, out_hbm, sv, wv, pv):
    cid = lax.axis_index("c")
    sid = lax.axis_index("s")
    wid = sid * SC_NC + cid
    row0 = wid * CPT
    pltpu.sync_copy(scoreT_hbm.at[pl.ds(row0, CPT)], sv)
    pltpu.sync_copy(wT_hbm.at[pl.ds(row0, CPT)], wv)

    acc = jnp.zeros((SCL,), jnp.float32)
    for c in range(CPT):
        ks = [sv[c, pl.ds(v * SCL, SCL)] for v in range(NV)]
        vs = [wv[c, pl.ds(v * SCL, SCL)] for v in range(NV)]
        _, tw = _sc_sort_col_kv(ks, vs)
        sw = _sc_sort_col_k(vs)
        for v in range(NV):
            d = sw[v] - tw[v]
            acc = acc + d * d
    pv[...] = acc
    pltpu.sync_copy(pv, out_hbm.at[wid])


_sc_sort = functools.partial(
    pl.kernel,
    out_type=jax.ShapeDtypeStruct((NW, SCL), jnp.float32),
    mesh=plsc.VectorSubcoreMesh(core_axis_name="c", subcore_axis_name="s"),
    compiler_params=pltpu.CompilerParams(needs_layout_passes=False),
    scratch_types=[
        pltpu.VMEM((CPT, IN), jnp.float32),
        pltpu.VMEM((CPT, IN), jnp.float32),
        pltpu.VMEM((SCL,), jnp.float32),
    ],
)(_sc_sort_body)


@jax.jit
def _run(x, t, w):
    xT = x.T
    tT = t.T
    wT = w.T
    scoreT = pl.pallas_call(
        _score_body,
        out_shape=jax.ShapeDtypeStruct((OUT, IN), jnp.float32),
    )(xT, tT)
    partials = _sc_sort(scoreT, wT)
    return jnp.sum(partials) / jnp.float32(IN * OUT)


def kernel(x, y, t, w):
    del y
    return _run(x, t, w)
